# baseline (speedup 1.0000x reference)
import math

import numpy as np
import jax
import jax.numpy as jnp
from jax import lax
from jax.experimental import pallas as pl

Hl, Wl = 14, 14
SCALE = 16
Hh, Wh = Hl * SCALE, Wl * SCALE
C = 96
R_MAX = 4
P = 256
NCELL = Hl * Wl
NCP = 224
CT = 128
FCOL = 16

_offs = [(dy, dx) for dy in range(-4, 5) for dx in range(-4, 5)
         if dy * dy + dx * dx <= R_MAX * R_MAX]
K = len(_offs)
KP = 56
_dyx = np.zeros((KP, 128), np.float32)
for _k, (_dy, _dx) in enumerate(_offs):
    _dyx[_k, 0] = _dy
    _dyx[_k, 1] = _dx
    _dyx[_k, 2] = _dy * _dy + _dx * _dx
DYX_NP = _dyx
SE_ROW = {(dy, dx): _offs.index((dy, dx))
          for dy in (-1, 0, 1) for dx in (-1, 0, 1)}


def _tc_body(dyx_ref, tab_ref, g_ref, o_ref):
    t = pl.program_id(0)
    r = t // Wl
    s = t - r * Wl

    dyc = dyx_ref[:, 0:1]
    dxc = dyx_ref[:, 1:2]
    rad2 = dyx_ref[:, 2:3]
    yn = jnp.clip(r + dyc.astype(jnp.int32), 0, Hl - 1)
    xn = jnp.clip(s + dxc.astype(jnp.int32), 0, Wl - 1)
    n_k = yn * Wl + xn
    n_iota = lax.broadcasted_iota(jnp.int32, (1, NCP), 1)
    onehot = (n_iota == n_k).astype(jnp.float32)
    win = lax.dot_general(onehot, tab_ref[...], (((1,), (0,)), ((), ())),
                          precision=lax.Precision.HIGHEST,
                          preferred_element_type=jnp.float32)

    col = lambda c: win[:, c : c + 1]
    cos_t, sin_t = col(0), col(1)
    i2sx, i2sy, i2sr = col(2), col(3), col(4)
    g0, g1, g2 = col(5), col(6), col(7)

    ii = lax.broadcasted_iota(jnp.int32, (1, P), 1)
    iq = ii // SCALE
    jq = ii - iq * SCALE
    yv = (iq + r * SCALE).astype(jnp.float32)
    xv = (jq + s * SCALE).astype(jnp.float32)

    cy = (yn.astype(jnp.float32) + 0.5) * SCALE - 0.5
    cx = (xn.astype(jnp.float32) + 0.5) * SCALE - 0.5
    dy = yv - cy
    dx = xv - cx
    xp = dx * cos_t + dy * sin_t
    yp = dy * cos_t - dx * sin_t
    log_w = -(xp * xp * i2sx + yp * yp * i2sy)

    gh0 = g_ref[0, 0:1, :]
    gh1 = g_ref[0, 1:2, :]
    gh2 = g_ref[0, 2:3, :]
    d0 = gh0 - g0
    d1 = gh1 - g1
    d2 = gh2 - g2
    log_w = log_w - (d0 * d0 + d1 * d1 + d2 * d2) * i2sr

    def se(a, b):
        row = SE_ROW[(a - 1, b - 1)]
        return win[row : row + 1, 8:9]
    ilo = iq < (SCALE // 2)
    jlo = jq < (SCALE // 2)
    rlo0 = jnp.where(jlo, se(0, 0), se(0, 1))
    rlo1 = jnp.where(jlo, se(1, 0), se(1, 1))
    rlo2 = jnp.where(jlo, se(2, 0), se(2, 1))
    rhi0 = jnp.where(jlo, se(0, 1), se(0, 2))
    rhi1 = jnp.where(jlo, se(1, 1), se(1, 2))
    rhi2 = jnp.where(jlo, se(2, 1), se(2, 2))
    v00 = jnp.where(ilo, rlo0, rlo1)
    v10 = jnp.where(ilo, rlo1, rlo2)
    v01 = jnp.where(ilo, rhi0, rhi1)
    v11 = jnp.where(ilo, rhi1, rhi2)
    i_f = iq.astype(jnp.float32)
    j_f = jq.astype(jnp.float32)
    ti = jnp.where(ilo, i_f + 8.5, i_f - 7.5) * (1.0 / SCALE)
    tj = jnp.where(jlo, j_f + 8.5, j_f - 7.5) * (1.0 / SCALE)
    se_hr = (1 - ti) * ((1 - tj) * v00 + tj * v01) + ti * ((1 - tj) * v10 + tj * v11)
    r_map = jnp.clip(jnp.ceil(2.0 * se_hr), 1.0, float(R_MAX))
    r2 = r_map * r_map

    kk = lax.broadcasted_iota(jnp.int32, (KP, 1), 0)
    valid = (kk < K) & (rad2 <= r2)
    log_w = jnp.where(valid, log_w, -1e30)

    m = jnp.max(log_w, axis=0, keepdims=True)
    w = jnp.exp(log_w - m)
    den = jnp.sum(w, axis=0, keepdims=True)

    feat = win[:, FCOL : FCOL + C]
    num = lax.dot_general(feat, w, (((0,), (0,)), ((), ())),
                          preferred_element_type=jnp.float32)
    o_ref[0] = num * (1.0 / jnp.maximum(den, 1e-8))


def _build_table(feat_lr, guide_hr, sx_raw, sy_raw, th_raw, sr_raw):
    sx = jnp.exp(sx_raw[0, 0])
    sy = jnp.exp(sy_raw[0, 0])
    th = math.pi * jnp.tanh(th_raw[0, 0])
    sr = jnp.exp(sr_raw[0, 0])
    sxm = jnp.maximum(sx, 1e-6)
    sym = jnp.maximum(sy, 1e-6)
    srm = jnp.maximum(sr, 1e-6)
    i2sx = 1.0 / (2.0 * sxm * sxm + 1e-8)
    i2sy = 1.0 / (2.0 * sym * sym + 1e-8)
    i2sr = 1.0 / (2.0 * srm * srm + 1e-8)
    se = jnp.maximum(sx, sy)

    gh = guide_hr[0]
    gl = 0.25 * (gh[:, 7::16, 7::16] + gh[:, 7::16, 8::16]
                 + gh[:, 8::16, 7::16] + gh[:, 8::16, 8::16])

    flat = lambda a: a.reshape(NCELL)
    cols = jnp.stack(
        [flat(jnp.cos(th)), flat(jnp.sin(th)), flat(i2sx), flat(i2sy),
         flat(i2sr), flat(gl[0]), flat(gl[1]), flat(gl[2]), flat(se)],
        axis=1)
    feat_flat = jnp.transpose(feat_lr[0], (1, 2, 0)).reshape(NCELL, C)
    table = jnp.concatenate(
        [cols, jnp.zeros((NCELL, FCOL - 9), jnp.float32), feat_flat,
         jnp.zeros((NCELL, CT - FCOL - C), jnp.float32)], axis=1)
    return jnp.concatenate(
        [table, jnp.zeros((NCP - NCELL, CT), jnp.float32)], axis=0)


def kernel(feat_lr, guide_hr, sx_raw, sy_raw, th_raw, sr_raw):
    table = _build_table(feat_lr, guide_hr, sx_raw, sy_raw, th_raw, sr_raw)

    guide_cm = (guide_hr[0].reshape(3, Hl, SCALE, Wl, SCALE)
                .transpose(1, 3, 0, 2, 4).reshape(NCELL, 3, P))

    out_cm = pl.pallas_call(
        _tc_body,
        grid=(NCELL,),
        in_specs=[
            pl.BlockSpec((KP, 128), lambda t: (0, 0)),
            pl.BlockSpec((NCP, CT), lambda t: (0, 0)),
            pl.BlockSpec((1, 3, P), lambda t: (t, 0, 0)),
        ],
        out_specs=pl.BlockSpec((1, C, P), lambda t: (t, 0, 0)),
        out_shape=jax.ShapeDtypeStruct((NCELL, C, P), jnp.float32),
    )(jnp.asarray(DYX_NP), table, guide_cm)

    return (out_cm.reshape(Hl, Wl, C, SCALE, SCALE)
            .transpose(2, 0, 3, 1, 4).reshape(1, C, Hh, Wh))


# speedup vs baseline: 1.0718x; 1.0718x over previous
import math

import numpy as np
import jax
import jax.numpy as jnp
from jax import lax
from jax.experimental import pallas as pl

Hl, Wl = 14, 14
SCALE = 16
Hh, Wh = Hl * SCALE, Wl * SCALE
C = 96
R_MAX = 4
P = 256
NCELL = Hl * Wl
CPB = 2
NCP = 224
CT = 128
FCOL = 16

_offs = [(dy, dx) for dy in range(-4, 5) for dx in range(-4, 5)
         if dy * dy + dx * dx <= R_MAX * R_MAX]
K = len(_offs)
KP = 56
_dyx = np.zeros((KP, 128), np.float32)
for _k, (_dy, _dx) in enumerate(_offs):
    _dyx[_k, 0] = _dy
    _dyx[_k, 1] = _dx
    _dyx[_k, 2] = _dy * _dy + _dx * _dx
DYX_NP = _dyx
SE_ROW = {(dy, dx): _offs.index((dy, dx))
          for dy in (-1, 0, 1) for dx in (-1, 0, 1)}


def _tc_body(dyx_ref, tab_ref, g_ref, o_ref):
    t = pl.program_id(0)
    for u in range(CPB):
        cell = CPB * t + u
        _tc_cell(cell, u, dyx_ref, tab_ref, g_ref, o_ref)


def _tc_cell(cell, u, dyx_ref, tab_ref, g_ref, o_ref):
    r = cell // Wl
    s = cell - r * Wl

    dyc = dyx_ref[:, 0:1]
    dxc = dyx_ref[:, 1:2]
    rad2 = dyx_ref[:, 2:3]
    yn = jnp.clip(r + dyc.astype(jnp.int32), 0, Hl - 1)
    xn = jnp.clip(s + dxc.astype(jnp.int32), 0, Wl - 1)
    n_k = yn * Wl + xn
    n_iota = lax.broadcasted_iota(jnp.int32, (1, NCP), 1)
    onehot = (n_iota == n_k).astype(jnp.float32)
    win = lax.dot_general(onehot, tab_ref[...], (((1,), (0,)), ((), ())),
                          precision=lax.Precision.HIGHEST,
                          preferred_element_type=jnp.float32)

    col = lambda c: win[:, c : c + 1]
    cos_t, sin_t = col(0), col(1)
    i2sx, i2sy, i2sr = col(2), col(3), col(4)
    g0, g1, g2 = col(5), col(6), col(7)

    ii = lax.broadcasted_iota(jnp.int32, (1, P), 1)
    iq = ii // SCALE
    jq = ii - iq * SCALE
    yv = (iq + r * SCALE).astype(jnp.float32)
    xv = (jq + s * SCALE).astype(jnp.float32)

    cy = (yn.astype(jnp.float32) + 0.5) * SCALE - 0.5
    cx = (xn.astype(jnp.float32) + 0.5) * SCALE - 0.5
    dy = yv - cy
    dx = xv - cx
    xp = dx * cos_t + dy * sin_t
    yp = dy * cos_t - dx * sin_t
    log_w = -(xp * xp * i2sx + yp * yp * i2sy)

    gh0 = g_ref[u, 0:1, :]
    gh1 = g_ref[u, 1:2, :]
    gh2 = g_ref[u, 2:3, :]
    d0 = gh0 - g0
    d1 = gh1 - g1
    d2 = gh2 - g2
    log_w = log_w - (d0 * d0 + d1 * d1 + d2 * d2) * i2sr

    def se(a, b):
        row = SE_ROW[(a - 1, b - 1)]
        return win[row : row + 1, 8:9]
    ilo = iq < (SCALE // 2)
    jlo = jq < (SCALE // 2)
    rlo0 = jnp.where(jlo, se(0, 0), se(0, 1))
    rlo1 = jnp.where(jlo, se(1, 0), se(1, 1))
    rlo2 = jnp.where(jlo, se(2, 0), se(2, 1))
    rhi0 = jnp.where(jlo, se(0, 1), se(0, 2))
    rhi1 = jnp.where(jlo, se(1, 1), se(1, 2))
    rhi2 = jnp.where(jlo, se(2, 1), se(2, 2))
    v00 = jnp.where(ilo, rlo0, rlo1)
    v10 = jnp.where(ilo, rlo1, rlo2)
    v01 = jnp.where(ilo, rhi0, rhi1)
    v11 = jnp.where(ilo, rhi1, rhi2)
    i_f = iq.astype(jnp.float32)
    j_f = jq.astype(jnp.float32)
    ti = jnp.where(ilo, i_f + 8.5, i_f - 7.5) * (1.0 / SCALE)
    tj = jnp.where(jlo, j_f + 8.5, j_f - 7.5) * (1.0 / SCALE)
    se_hr = (1 - ti) * ((1 - tj) * v00 + tj * v01) + ti * ((1 - tj) * v10 + tj * v11)
    r_map = jnp.clip(jnp.ceil(2.0 * se_hr), 1.0, float(R_MAX))
    r2 = r_map * r_map

    kk = lax.broadcasted_iota(jnp.int32, (KP, 1), 0)
    valid = (kk < K) & (rad2 <= r2)
    log_w = jnp.where(valid, log_w, -1e30)

    m = jnp.max(log_w, axis=0, keepdims=True)
    w = jnp.exp(log_w - m)
    den = jnp.sum(w, axis=0, keepdims=True)

    feat = win[:, FCOL : FCOL + C]
    num = lax.dot_general(feat, w, (((0,), (0,)), ((), ())),
                          preferred_element_type=jnp.float32)
    o_ref[u] = num * (1.0 / jnp.maximum(den, 1e-8))


def _build_table(feat_lr, guide_hr, sx_raw, sy_raw, th_raw, sr_raw):
    sx = jnp.exp(sx_raw[0, 0])
    sy = jnp.exp(sy_raw[0, 0])
    th = math.pi * jnp.tanh(th_raw[0, 0])
    sr = jnp.exp(sr_raw[0, 0])
    sxm = jnp.maximum(sx, 1e-6)
    sym = jnp.maximum(sy, 1e-6)
    srm = jnp.maximum(sr, 1e-6)
    i2sx = 1.0 / (2.0 * sxm * sxm + 1e-8)
    i2sy = 1.0 / (2.0 * sym * sym + 1e-8)
    i2sr = 1.0 / (2.0 * srm * srm + 1e-8)
    se = jnp.maximum(sx, sy)

    gh = guide_hr[0]
    gl = 0.25 * (gh[:, 7::16, 7::16] + gh[:, 7::16, 8::16]
                 + gh[:, 8::16, 7::16] + gh[:, 8::16, 8::16])

    flat = lambda a: a.reshape(NCELL)
    cols = jnp.stack(
        [flat(jnp.cos(th)), flat(jnp.sin(th)), flat(i2sx), flat(i2sy),
         flat(i2sr), flat(gl[0]), flat(gl[1]), flat(gl[2]), flat(se)],
        axis=1)
    feat_flat = jnp.transpose(feat_lr[0], (1, 2, 0)).reshape(NCELL, C)
    table = jnp.concatenate(
        [cols, jnp.zeros((NCELL, FCOL - 9), jnp.float32), feat_flat,
         jnp.zeros((NCELL, CT - FCOL - C), jnp.float32)], axis=1)
    return jnp.concatenate(
        [table, jnp.zeros((NCP - NCELL, CT), jnp.float32)], axis=0)


def kernel(feat_lr, guide_hr, sx_raw, sy_raw, th_raw, sr_raw):
    table = _build_table(feat_lr, guide_hr, sx_raw, sy_raw, th_raw, sr_raw)

    guide_cm = (guide_hr[0].reshape(3, Hl, SCALE, Wl, SCALE)
                .transpose(1, 3, 0, 2, 4).reshape(NCELL, 3, P))

    out_cm = pl.pallas_call(
        _tc_body,
        grid=(NCELL // CPB,),
        in_specs=[
            pl.BlockSpec((KP, 128), lambda t: (0, 0)),
            pl.BlockSpec((NCP, CT), lambda t: (0, 0)),
            pl.BlockSpec((CPB, 3, P), lambda t: (t, 0, 0)),
        ],
        out_specs=pl.BlockSpec((CPB, C, P), lambda t: (t, 0, 0)),
        out_shape=jax.ShapeDtypeStruct((NCELL, C, P), jnp.float32),
    )(jnp.asarray(DYX_NP), table, guide_cm)

    return (out_cm.reshape(Hl, Wl, C, SCALE, SCALE)
            .transpose(2, 0, 3, 1, 4).reshape(1, C, Hh, Wh))


# speedup vs baseline: 1.2041x; 1.1234x over previous
import math

import numpy as np
import jax
import jax.numpy as jnp
from jax import lax
from jax.experimental import pallas as pl

Hl, Wl = 14, 14
SCALE = 16
Hh, Wh = Hl * SCALE, Wl * SCALE
C = 96
R_MAX = 4
P = 256
NCELL = Hl * Wl
CPB = 14
NCP = 224
CT = 128
FCOL = 16

_offs = [(dy, dx) for dy in range(-4, 5) for dx in range(-4, 5)
         if dy * dy + dx * dx <= R_MAX * R_MAX]
K = len(_offs)
KP = 56
_dyx = np.zeros((KP, 128), np.float32)
for _k, (_dy, _dx) in enumerate(_offs):
    _dyx[_k, 0] = _dy
    _dyx[_k, 1] = _dx
    _dyx[_k, 2] = _dy * _dy + _dx * _dx
DYX_NP = _dyx
SE_ROW = {(dy, dx): _offs.index((dy, dx))
          for dy in (-1, 0, 1) for dx in (-1, 0, 1)}


def _tc_body(dyx_ref, tab_ref, g_ref, o_ref):
    t = pl.program_id(0)
    for u in range(CPB):
        cell = CPB * t + u
        _tc_cell(cell, u, dyx_ref, tab_ref, g_ref, o_ref)


def _tc_cell(cell, u, dyx_ref, tab_ref, g_ref, o_ref):
    r = cell // Wl
    s = cell - r * Wl

    dyc = dyx_ref[:, 0:1]
    dxc = dyx_ref[:, 1:2]
    rad2 = dyx_ref[:, 2:3]
    yn = jnp.clip(r + dyc.astype(jnp.int32), 0, Hl - 1)
    xn = jnp.clip(s + dxc.astype(jnp.int32), 0, Wl - 1)
    n_k = yn * Wl + xn
    n_iota = lax.broadcasted_iota(jnp.int32, (1, NCP), 1)
    onehot = (n_iota == n_k).astype(jnp.float32)
    win = lax.dot_general(onehot, tab_ref[:, 0:FCOL], (((1,), (0,)), ((), ())),
                          precision=lax.Precision.HIGHEST,
                          preferred_element_type=jnp.float32)
    feat = lax.dot_general(onehot, tab_ref[:, FCOL : FCOL + C],
                           (((1,), (0,)), ((), ())),
                           preferred_element_type=jnp.float32)

    col = lambda c: win[:, c : c + 1]
    cos_t, sin_t = col(0), col(1)
    i2sx, i2sy, i2sr = col(2), col(3), col(4)
    g0, g1, g2 = col(5), col(6), col(7)

    ii = lax.broadcasted_iota(jnp.int32, (1, P), 1)
    iq = ii // SCALE
    jq = ii - iq * SCALE
    yv = (iq + r * SCALE).astype(jnp.float32)
    xv = (jq + s * SCALE).astype(jnp.float32)

    cy = (yn.astype(jnp.float32) + 0.5) * SCALE - 0.5
    cx = (xn.astype(jnp.float32) + 0.5) * SCALE - 0.5
    dy = yv - cy
    dx = xv - cx
    xp = dx * cos_t + dy * sin_t
    yp = dy * cos_t - dx * sin_t
    log_w = -(xp * xp * i2sx + yp * yp * i2sy)

    gh0 = g_ref[u, 0:1, :]
    gh1 = g_ref[u, 1:2, :]
    gh2 = g_ref[u, 2:3, :]
    d0 = gh0 - g0
    d1 = gh1 - g1
    d2 = gh2 - g2
    log_w = log_w - (d0 * d0 + d1 * d1 + d2 * d2) * i2sr

    def se(a, b):
        row = SE_ROW[(a - 1, b - 1)]
        return win[row : row + 1, 8:9]
    ilo = iq < (SCALE // 2)
    jlo = jq < (SCALE // 2)
    rlo0 = jnp.where(jlo, se(0, 0), se(0, 1))
    rlo1 = jnp.where(jlo, se(1, 0), se(1, 1))
    rlo2 = jnp.where(jlo, se(2, 0), se(2, 1))
    rhi0 = jnp.where(jlo, se(0, 1), se(0, 2))
    rhi1 = jnp.where(jlo, se(1, 1), se(1, 2))
    rhi2 = jnp.where(jlo, se(2, 1), se(2, 2))
    v00 = jnp.where(ilo, rlo0, rlo1)
    v10 = jnp.where(ilo, rlo1, rlo2)
    v01 = jnp.where(ilo, rhi0, rhi1)
    v11 = jnp.where(ilo, rhi1, rhi2)
    i_f = iq.astype(jnp.float32)
    j_f = jq.astype(jnp.float32)
    ti = jnp.where(ilo, i_f + 8.5, i_f - 7.5) * (1.0 / SCALE)
    tj = jnp.where(jlo, j_f + 8.5, j_f - 7.5) * (1.0 / SCALE)
    se_hr = (1 - ti) * ((1 - tj) * v00 + tj * v01) + ti * ((1 - tj) * v10 + tj * v11)
    r_map = jnp.clip(jnp.ceil(2.0 * se_hr), 1.0, float(R_MAX))
    r2 = r_map * r_map

    kk = lax.broadcasted_iota(jnp.int32, (KP, 1), 0)
    valid = (kk < K) & (rad2 <= r2)
    log_w = jnp.where(valid, log_w, -1e30)

    m = jnp.max(log_w, axis=0, keepdims=True)
    w = jnp.exp(log_w - m)
    den = jnp.sum(w, axis=0, keepdims=True)

    num = lax.dot_general(feat, w, (((0,), (0,)), ((), ())),
                          preferred_element_type=jnp.float32)
    o_ref[u] = num * (1.0 / jnp.maximum(den, 1e-8))


def _build_table(feat_lr, guide_hr, sx_raw, sy_raw, th_raw, sr_raw):
    sx = jnp.exp(sx_raw[0, 0])
    sy = jnp.exp(sy_raw[0, 0])
    th = math.pi * jnp.tanh(th_raw[0, 0])
    sr = jnp.exp(sr_raw[0, 0])
    sxm = jnp.maximum(sx, 1e-6)
    sym = jnp.maximum(sy, 1e-6)
    srm = jnp.maximum(sr, 1e-6)
    i2sx = 1.0 / (2.0 * sxm * sxm + 1e-8)
    i2sy = 1.0 / (2.0 * sym * sym + 1e-8)
    i2sr = 1.0 / (2.0 * srm * srm + 1e-8)
    se = jnp.maximum(sx, sy)

    gh = guide_hr[0]
    gl = 0.25 * (gh[:, 7::16, 7::16] + gh[:, 7::16, 8::16]
                 + gh[:, 8::16, 7::16] + gh[:, 8::16, 8::16])

    flat = lambda a: a.reshape(NCELL)
    cols = jnp.stack(
        [flat(jnp.cos(th)), flat(jnp.sin(th)), flat(i2sx), flat(i2sy),
         flat(i2sr), flat(gl[0]), flat(gl[1]), flat(gl[2]), flat(se)],
        axis=1)
    feat_flat = jnp.transpose(feat_lr[0], (1, 2, 0)).reshape(NCELL, C)
    table = jnp.concatenate(
        [cols, jnp.zeros((NCELL, FCOL - 9), jnp.float32), feat_flat,
         jnp.zeros((NCELL, CT - FCOL - C), jnp.float32)], axis=1)
    return jnp.concatenate(
        [table, jnp.zeros((NCP - NCELL, CT), jnp.float32)], axis=0)


def kernel(feat_lr, guide_hr, sx_raw, sy_raw, th_raw, sr_raw):
    table = _build_table(feat_lr, guide_hr, sx_raw, sy_raw, th_raw, sr_raw)

    guide_cm = (guide_hr[0].reshape(3, Hl, SCALE, Wl, SCALE)
                .transpose(1, 3, 0, 2, 4).reshape(NCELL, 3, P))

    out_cm = pl.pallas_call(
        _tc_body,
        grid=(NCELL // CPB,),
        in_specs=[
            pl.BlockSpec((KP, 128), lambda t: (0, 0)),
            pl.BlockSpec((NCP, CT), lambda t: (0, 0)),
            pl.BlockSpec((CPB, 3, P), lambda t: (t, 0, 0)),
        ],
        out_specs=pl.BlockSpec((CPB, C, P), lambda t: (t, 0, 0)),
        out_shape=jax.ShapeDtypeStruct((NCELL, C, P), jnp.float32),
    )(jnp.asarray(DYX_NP), table, guide_cm)

    return (out_cm.reshape(Hl, Wl, C, SCALE, SCALE)
            .transpose(2, 0, 3, 1, 4).reshape(1, C, Hh, Wh))


# speedup vs baseline: 1.2052x; 1.0009x over previous
import math

import numpy as np
import jax
import jax.numpy as jnp
from jax import lax
from jax.experimental import pallas as pl

Hl, Wl = 14, 14
SCALE = 16
Hh, Wh = Hl * SCALE, Wl * SCALE
C = 96
R_MAX = 4
P = 256
NCELL = Hl * Wl
CPB = 14
NCP = 224
CT = 112
FCOL = 16

_offs = [(dy, dx) for dy in range(-4, 5) for dx in range(-4, 5)
         if dy * dy + dx * dx <= R_MAX * R_MAX]
K = len(_offs)
KP = 56
_dyx = np.zeros((KP, 128), np.float32)
for _k, (_dy, _dx) in enumerate(_offs):
    _dyx[_k, 0] = _dy
    _dyx[_k, 1] = _dx
    _dyx[_k, 2] = _dy * _dy + _dx * _dx
DYX_NP = _dyx
SE_ROW = {(dy, dx): _offs.index((dy, dx))
          for dy in (-1, 0, 1) for dx in (-1, 0, 1)}


def _tc_body(dyx_ref, tab_ref, g_ref, o_ref):
    t = pl.program_id(0)
    for u in range(CPB):
        cell = CPB * t + u
        _tc_cell(cell, u, dyx_ref, tab_ref, g_ref, o_ref)


def _tc_cell(cell, u, dyx_ref, tab_ref, g_ref, o_ref):
    r = cell // Wl
    s = cell - r * Wl

    dyc = dyx_ref[:, 0:1]
    dxc = dyx_ref[:, 1:2]
    rad2 = dyx_ref[:, 2:3]
    yn = jnp.clip(r + dyc.astype(jnp.int32), 0, Hl - 1)
    xn = jnp.clip(s + dxc.astype(jnp.int32), 0, Wl - 1)
    n_k = yn * Wl + xn
    n_iota = lax.broadcasted_iota(jnp.int32, (1, NCP), 1)
    onehot = (n_iota == n_k).astype(jnp.float32)
    win = lax.dot_general(onehot, tab_ref[:, 0:FCOL], (((1,), (0,)), ((), ())),
                          precision=lax.Precision.HIGHEST,
                          preferred_element_type=jnp.float32)
    feat = lax.dot_general(onehot, tab_ref[:, FCOL : FCOL + C],
                           (((1,), (0,)), ((), ())),
                           preferred_element_type=jnp.float32)

    col = lambda c: win[:, c : c + 1]
    cos_t, sin_t = col(0), col(1)
    i2sx, i2sy, i2sr = col(2), col(3), col(4)
    g0, g1, g2 = col(5), col(6), col(7)

    ii = lax.broadcasted_iota(jnp.int32, (1, P), 1)
    iq = ii // SCALE
    jq = ii - iq * SCALE
    yv = (iq + r * SCALE).astype(jnp.float32)
    xv = (jq + s * SCALE).astype(jnp.float32)

    cy = (yn.astype(jnp.float32) + 0.5) * SCALE - 0.5
    cx = (xn.astype(jnp.float32) + 0.5) * SCALE - 0.5
    dy = yv - cy
    dx = xv - cx
    xp = dx * cos_t + dy * sin_t
    yp = dy * cos_t - dx * sin_t
    log_w = -(xp * xp * i2sx + yp * yp * i2sy)

    gh0 = g_ref[u, 0:1, :]
    gh1 = g_ref[u, 1:2, :]
    gh2 = g_ref[u, 2:3, :]
    d0 = gh0 - g0
    d1 = gh1 - g1
    d2 = gh2 - g2
    log_w = log_w - (d0 * d0 + d1 * d1 + d2 * d2) * i2sr

    def se(a, b):
        row = SE_ROW[(a - 1, b - 1)]
        return win[row : row + 1, 8:9]
    ilo = iq < (SCALE // 2)
    jlo = jq < (SCALE // 2)
    rlo0 = jnp.where(jlo, se(0, 0), se(0, 1))
    rlo1 = jnp.where(jlo, se(1, 0), se(1, 1))
    rlo2 = jnp.where(jlo, se(2, 0), se(2, 1))
    rhi0 = jnp.where(jlo, se(0, 1), se(0, 2))
    rhi1 = jnp.where(jlo, se(1, 1), se(1, 2))
    rhi2 = jnp.where(jlo, se(2, 1), se(2, 2))
    v00 = jnp.where(ilo, rlo0, rlo1)
    v10 = jnp.where(ilo, rlo1, rlo2)
    v01 = jnp.where(ilo, rhi0, rhi1)
    v11 = jnp.where(ilo, rhi1, rhi2)
    i_f = iq.astype(jnp.float32)
    j_f = jq.astype(jnp.float32)
    ti = jnp.where(ilo, i_f + 8.5, i_f - 7.5) * (1.0 / SCALE)
    tj = jnp.where(jlo, j_f + 8.5, j_f - 7.5) * (1.0 / SCALE)
    se_hr = (1 - ti) * ((1 - tj) * v00 + tj * v01) + ti * ((1 - tj) * v10 + tj * v11)
    r_map = jnp.clip(jnp.ceil(2.0 * se_hr), 1.0, float(R_MAX))
    r2 = r_map * r_map

    kk = lax.broadcasted_iota(jnp.int32, (KP, 1), 0)
    valid = (kk < K) & (rad2 <= r2)
    log_w = jnp.where(valid, log_w, -1e30)

    m = jnp.max(log_w, axis=0, keepdims=True)
    w = jnp.exp(log_w - m)
    den = jnp.sum(w, axis=0, keepdims=True)

    num = lax.dot_general(feat, w, (((0,), (0,)), ((), ())),
                          preferred_element_type=jnp.float32)
    o_ref[u] = num * (1.0 / jnp.maximum(den, 1e-8))


def _build_table(feat_lr, guide_hr, sx_raw, sy_raw, th_raw, sr_raw):
    sx = jnp.exp(sx_raw[0, 0])
    sy = jnp.exp(sy_raw[0, 0])
    th = math.pi * jnp.tanh(th_raw[0, 0])
    sr = jnp.exp(sr_raw[0, 0])
    sxm = jnp.maximum(sx, 1e-6)
    sym = jnp.maximum(sy, 1e-6)
    srm = jnp.maximum(sr, 1e-6)
    i2sx = 1.0 / (2.0 * sxm * sxm + 1e-8)
    i2sy = 1.0 / (2.0 * sym * sym + 1e-8)
    i2sr = 1.0 / (2.0 * srm * srm + 1e-8)
    se = jnp.maximum(sx, sy)

    gh = guide_hr[0]
    gl = 0.25 * (gh[:, 7::16, 7::16] + gh[:, 7::16, 8::16]
                 + gh[:, 8::16, 7::16] + gh[:, 8::16, 8::16])

    planes = jnp.concatenate(
        [jnp.stack([jnp.cos(th), jnp.sin(th), i2sx, i2sy, i2sr,
                    gl[0], gl[1], gl[2], se], axis=0),
         jnp.zeros((FCOL - 9, Hl, Wl), jnp.float32),
         feat_lr[0]], axis=0).reshape(FCOL + C, NCELL)
    table = planes.T
    return jnp.concatenate(
        [table, jnp.zeros((NCP - NCELL, FCOL + C), jnp.float32)], axis=0)


def kernel(feat_lr, guide_hr, sx_raw, sy_raw, th_raw, sr_raw):
    table = _build_table(feat_lr, guide_hr, sx_raw, sy_raw, th_raw, sr_raw)

    guide_cm = (guide_hr[0].reshape(3, Hl, SCALE, Wl, SCALE)
                .transpose(1, 3, 0, 2, 4).reshape(NCELL, 3, P))

    out_cm = pl.pallas_call(
        _tc_body,
        grid=(NCELL // CPB,),
        in_specs=[
            pl.BlockSpec((KP, 128), lambda t: (0, 0)),
            pl.BlockSpec((NCP, CT), lambda t: (0, 0)),
            pl.BlockSpec((CPB, 3, P), lambda t: (t, 0, 0)),
        ],
        out_specs=pl.BlockSpec((CPB, C, P), lambda t: (t, 0, 0)),
        out_shape=jax.ShapeDtypeStruct((NCELL, C, P), jnp.float32),
    )(jnp.asarray(DYX_NP), table, guide_cm)

    return (out_cm.reshape(Hl, Wl, C, SCALE, SCALE)
            .transpose(2, 0, 3, 1, 4).reshape(1, C, Hh, Wh))
